# Initial kernel scaffold; baseline (speedup 1.0000x reference)
import jax
import jax.numpy as jnp
from jax.experimental import pallas as pl


def kernel(xyz, params):
    raise NotImplementedError("write your pallas kernel here")


# speedup vs baseline: 9.7079x; 9.7079x over previous
import functools

import jax
import jax.numpy as jnp
from jax import lax
from jax.experimental import pallas as pl
from jax.experimental.pallas import tpu as pltpu
from jax.experimental.pallas import tpu_sc as plsc

_BN_EPS = 1e-5
_B = 16
_N = 2048


def _fps_body(p_ref, o_ref, *, n_steps):
    bsz, _, n = p_ref.shape
    px = p_ref[:, 0, :]
    py = p_ref[:, 1, :]
    pz = p_ref[:, 2, :]
    jidx = lax.broadcasted_iota(jnp.int32, (bsz, n), 1)
    zero5 = jnp.zeros((bsz, 5), jnp.float32)

    def step(t, carry):
        dist, far = carry
        m = (jidx == far).astype(jnp.float32)
        c0 = jnp.sum(m * px, axis=1, keepdims=True)
        c1 = jnp.sum(m * py, axis=1, keepdims=True)
        c2 = jnp.sum(m * pz, axis=1, keepdims=True)
        o_ref[pl.ds(t, 1)] = jnp.concatenate([c0, c1, c2, zero5], axis=1)[None]
        d0 = px - c0
        d1 = py - c1
        d2 = pz - c2
        d = d0 * d0 + d1 * d1 + d2 * d2
        nd = jnp.minimum(dist, d)
        nf = jnp.argmax(nd, axis=1, keepdims=True).astype(jnp.int32)
        return nd, nf

    init = (jnp.full((bsz, n), 1e10, jnp.float32), jnp.zeros((bsz, 1), jnp.int32))
    lax.fori_loop(0, n_steps, step, init)


def _fps(pts8, n_steps):
    bsz, _, n = pts8.shape
    out = pl.pallas_call(
        functools.partial(_fps_body, n_steps=n_steps),
        out_shape=jax.ShapeDtypeStruct((n_steps, bsz, 8), jnp.float32),
    )(pts8)
    return jnp.transpose(out, (1, 0, 2))


def _cumsum_lanes(x, n):
    s = 1
    while s < n:
        x = x + jnp.pad(x, ((0, 0), (s, 0)))[:, :n]
        s *= 2
    return x


def _bq_body(c_ref, p_ref, o_ref, *, r2, K, n):
    b = pl.program_id(0)
    c = c_ref[0]
    p = p_ref[0]
    c0, c1, c2 = c[:, 0:1], c[:, 1:2], c[:, 2:3]
    p0, p1, p2 = p[0:1], p[1:2], p[2:3]
    dot = jnp.dot(
        c, p, precision=jax.lax.Precision.DEFAULT,
        preferred_element_type=jnp.float32,
    )
    a2 = c0 * c0 + c1 * c1 + c2 * c2
    b2 = p0 * p0 + p1 * p1 + p2 * p2
    d = (-2.0 * dot + a2) + b2
    maskf = (d <= r2).astype(jnp.float32)
    rkm = maskf * _cumsum_lanes(maskf, n)
    ji = lax.broadcasted_iota(jnp.int32, d.shape, 1).astype(jnp.float32) + 1.0
    cols = []
    for k in range(K):
        sel = rkm == float(k + 1)
        cols.append(jnp.sum(jnp.where(sel, ji, 0.0), axis=1, keepdims=True))
    idxf = jnp.concatenate(cols, axis=1)
    first = idxf[:, 0:1]
    fill = jnp.where(first > 0.5, first - 1.0, float(n - 1))
    idxv = jnp.where(idxf > 0.5, idxf - 1.0, fill)
    idxv = jnp.clip(idxv, 0.0, float(n - 1))
    o_ref[0] = idxv.astype(jnp.int32) + b * n


def _ball_query(cent, pts8, radius, K, s_blk):
    bsz, s, _ = cent.shape
    n = pts8.shape[2]
    return pl.pallas_call(
        functools.partial(_bq_body, r2=radius * radius, K=K, n=n),
        grid=(bsz, s // s_blk),
        in_specs=[
            pl.BlockSpec((1, s_blk, 8), lambda b, i: (b, i, 0)),
            pl.BlockSpec((1, 8, n), lambda b, i: (b, 0, 0)),
        ],
        out_specs=pl.BlockSpec((1, s_blk, K), lambda b, i: (b, i, 0)),
        out_shape=jax.ShapeDtypeStruct((bsz, s, K), jnp.int32),
    )(cent, pts8)


def _gather_rows(table, idx):
    v, dd = table.shape
    btot = idx.shape[0]
    info = plsc.get_sparse_core_info()
    nw = info.num_cores * info.num_subcores
    nc = info.num_cores
    bpw = btot // nw
    ch = 128
    n_it = bpw // ch
    mesh = plsc.VectorSubcoreMesh(core_axis_name="c", subcore_axis_name="s")

    @functools.partial(
        pl.kernel,
        out_type=jax.ShapeDtypeStruct((btot, dd), jnp.float32),
        mesh=mesh,
        scratch_types=[
            pltpu.VMEM((ch,), jnp.int32),
            pltpu.VMEM((ch, dd), jnp.float32),
            pltpu.SemaphoreType.DMA,
        ],
    )
    def k(table_hbm, idx_hbm, out_hbm, idx_v, rows_v, sem):
        wid = lax.axis_index("s") * nc + lax.axis_index("c")
        base = wid * bpw

        def body(i, _):
            off = base + i * ch
            pltpu.sync_copy(idx_hbm.at[pl.ds(off, ch)], idx_v)
            pltpu.async_copy(table_hbm.at[idx_v], rows_v, sem).wait()
            pltpu.sync_copy(rows_v, out_hbm.at[pl.ds(off, ch)])
            return 0

        lax.fori_loop(0, n_it, body, 0)

    return k(table, idx)


def _lin_body(*refs, has_aux, has_corr, K, R):
    it = iter(refs)
    x_ref = next(it)
    w_ref = next(it)
    b_ref = next(it)
    aux_ref = next(it) if has_aux else None
    c_ref = next(it) if has_corr else None
    y_ref = next(it)

    xv = x_ref[...]
    if has_aux:
        xv = jnp.maximum(
            (xv - aux_ref[0:1, :]) / aux_ref[1:2, :] * aux_ref[2:3, :]
            + aux_ref[3:4, :],
            0.0,
        )
    if has_corr:
        sb = R // K
        din = xv.shape[1]
        cb = jnp.broadcast_to(c_ref[...][:, None, :], (sb, K, din)).reshape(R, din)
        xv = xv - cb
    w = w_ref[...]
    y_ref[...] = (
        jnp.dot(xv, w, precision=jax.lax.Precision.DEFAULT,
                preferred_element_type=jnp.float32) + b_ref[0:1, :]
    )


def _linear(x, w_t, bias, aux=None, corr=None, K=1, R=2048):
    m, din = x.shape
    dout = w_t.shape[1]
    inputs = [x, w_t, bias.reshape(1, dout)]
    specs = [
        pl.BlockSpec((R, din), lambda i: (i, 0)),
        pl.BlockSpec((din, dout), lambda i: (0, 0)),
        pl.BlockSpec((1, dout), lambda i: (0, 0)),
    ]
    if aux is not None:
        inputs.append(aux)
        specs.append(pl.BlockSpec((8, din), lambda i: (0, 0)))
    if corr is not None:
        inputs.append(corr)
        specs.append(pl.BlockSpec((R // K, din), lambda i: (i, 0)))
    return pl.pallas_call(
        functools.partial(
            _lin_body, has_aux=aux is not None, has_corr=corr is not None, K=K, R=R
        ),
        grid=(m // R,),
        in_specs=specs,
        out_specs=pl.BlockSpec((R, dout), lambda i: (i, 0)),
        out_shape=jax.ShapeDtypeStruct((m, dout), jnp.float32),
    )(*inputs)


def _bn_rows(y, bshape, g, bt):
    ym = y.reshape(bshape + (y.shape[1],))
    mean = ym.mean(axis=(0, 1, 2))
    sv = jnp.sqrt(ym.var(axis=(0, 1, 2)) + _BN_EPS)
    return jnp.concatenate(
        [mean[None], sv[None], g[None], bt[None],
         jnp.zeros((4, mean.shape[0]), jnp.float32)],
        axis=0,
    )


def _maxpool_body(y_ref, aux_ref, o_ref):
    y = y_ref[...]
    m = aux_ref[0:1, :][None]
    sv = aux_ref[1:2, :][None]
    g = aux_ref[2:3, :][None]
    bt = aux_ref[3:4, :][None]
    z = jnp.maximum((y - m) / sv * g + bt, 0.0)
    o_ref[...] = jnp.max(z, axis=1)


def _maxpool(y3, aux, s_blk):
    mrows, kk, d = y3.shape
    return pl.pallas_call(
        _maxpool_body,
        grid=(mrows // s_blk,),
        in_specs=[
            pl.BlockSpec((s_blk, kk, d), lambda i: (i, 0, 0)),
            pl.BlockSpec((8, d), lambda i: (0, 0)),
        ],
        out_specs=pl.BlockSpec((s_blk, d), lambda i: (i, 0)),
        out_shape=jax.ShapeDtypeStruct((mrows, d), jnp.float32),
    )(y3, aux)


def _bn_rows_fc(y, g, bt):
    mean = y.mean(axis=0)
    sv = jnp.sqrt(y.var(axis=0) + _BN_EPS)
    return jnp.concatenate(
        [mean[None], sv[None], g[None], bt[None],
         jnp.zeros((4, mean.shape[0]), jnp.float32)],
        axis=0,
    )


def _fc_head(x, layers):
    (w1, b1, g1, t1), (w2, b2, g2, t2), (w3, b3, g3, t3), (w4, b4, g4, t4) = layers
    y = _linear(x, jnp.transpose(w1), b1, R=x.shape[0])
    a = _bn_rows_fc(y, g1, t1)
    y = _linear(y, jnp.transpose(w2), b2, aux=a, R=x.shape[0])
    a = _bn_rows_fc(y, g2, t2)
    latent = jnp.maximum((y - a[0]) / a[1] * a[2] + a[3], 0.0)
    y = _linear(y, jnp.transpose(w3), b3, aux=a, R=x.shape[0])
    a = _bn_rows_fc(y, g3, t3)
    y = _linear(y, jnp.transpose(w4), b4, aux=a, R=x.shape[0])
    a = _bn_rows_fc(y, g4, t4)
    return latent, y, a


def _dout_body(x_ref, w_ref, b_ref, aux_ref, o_ref):
    a = aux_ref
    xv = jnp.maximum(
        (x_ref[...] - a[0:1, :]) / a[1:2, :] * a[2:3, :] + a[3:4, :], 0.0
    )
    o_ref[...] = (
        jnp.dot(xv, w_ref[...], precision=jax.lax.Precision.DEFAULT,
                preferred_element_type=jnp.float32)
        + b_ref[0:1, :]
    )


def _dout(x, w, b, aux, c_blk=1024):
    m, din = x.shape
    dall = w.shape[0]
    wt = jnp.transpose(w)
    return pl.pallas_call(
        _dout_body,
        grid=(dall // c_blk,),
        in_specs=[
            pl.BlockSpec((m, din), lambda i: (0, 0)),
            pl.BlockSpec((din, c_blk), lambda i: (0, i)),
            pl.BlockSpec((1, c_blk), lambda i: (0, i)),
            pl.BlockSpec((8, din), lambda i: (0, 0)),
        ],
        out_specs=pl.BlockSpec((m, c_blk), lambda i: (0, i)),
        out_shape=jax.ShapeDtypeStruct((m, dall), jnp.float32),
    )(x, wt, b.reshape(1, dall), aux)


def _pad_wt(w, din_pad):
    wt = jnp.transpose(w)
    return jnp.pad(wt, ((0, din_pad - wt.shape[0]), (0, 0)))


def _sa_mlp(g, layers, din_pad, corr, K, bshape, R, pool_blk):
    mrows = bshape[0] * bshape[1]
    (w1, b1, g1, t1), (w2, b2, g2, t2), (w3, b3, g3, t3) = layers
    y = _linear(g, _pad_wt(w1, din_pad), b1, corr=corr, K=K, R=R)
    aux = _bn_rows(y, bshape, g1, t1)
    y = _linear(y, jnp.transpose(w2), b2, aux=aux, R=R)
    aux = _bn_rows(y, bshape, g2, t2)
    y = _linear(y, jnp.transpose(w3), b3, aux=aux, R=R)
    aux = _bn_rows(y, bshape, g3, t3)
    return _maxpool(y.reshape(mrows, K, w3.shape[0]), aux, pool_blk)


def kernel(xyz, params):
    bsz = xyz.shape[0]

    pts8 = jnp.concatenate(
        [xyz[:, :3, :], jnp.zeros((bsz, 5, _N), jnp.float32)], axis=1
    )
    table1 = jnp.pad(jnp.transpose(xyz, (0, 2, 1)), ((0, 0), (0, 0), (0, 122)))
    table1 = table1.reshape(bsz * _N, 128)

    c1 = _fps(pts8, 512)
    idx1 = _ball_query(c1, pts8, 0.2, 32, 128)
    g1 = _gather_rows(table1, idx1.reshape(-1))
    c1_pad = jnp.pad(c1.reshape(bsz * 512, 8), ((0, 0), (0, 120)))
    l1p = _sa_mlp(
        g1, params["sa1"], 128, c1_pad, 32, (bsz, 512, 32), 2048, 128
    )

    p2 = jnp.transpose(c1, (0, 2, 1))
    c2 = _fps(p2, 128)
    idx2 = _ball_query(c2, p2, 0.4, 64, 128)
    table2 = jnp.concatenate(
        [
            c1[:, :, 0:3],
            l1p.reshape(bsz, 512, 128),
            jnp.zeros((bsz, 512, 125), jnp.float32),
        ],
        axis=2,
    ).reshape(bsz * 512, 256)
    g2 = _gather_rows(table2, idx2.reshape(-1))
    c2_pad = jnp.pad(c2.reshape(bsz * 128, 8), ((0, 0), (0, 248)))
    l2p = _sa_mlp(
        g2, params["sa2"], 256, c2_pad, 64, (bsz, 128, 64), 2048, 32
    )

    x3 = jnp.concatenate(
        [
            c2[:, :, 0:3],
            l2p.reshape(bsz, 128, 256),
            jnp.zeros((bsz, 128, 125), jnp.float32),
        ],
        axis=2,
    ).reshape(bsz * 128, 384)
    l3p = _sa_mlp(x3, params["sa3"], 384, None, 128, (bsz, 1, 128), 512, 8)

    latent, h_pre, h_aux = _fc_head(
        l3p, [params["fc1"], params["fc2"], params["dfc1"], params["dfc2"]]
    )
    w_out, b_out = params["dout"]
    out = _dout(h_pre, w_out, b_out, h_aux).reshape(bsz, _N, 3)
    return (out, latent)
